# Initial kernel scaffold; baseline (speedup 1.0000x reference)
import jax
import jax.numpy as jnp
from jax.experimental import pallas as pl


def kernel(x, smoothed_vert_pos, edge_index, W_gcn1, b_gcn1, W_gcn2, b_gcn2, W_ih, W_hh, b_ih, b_hh, W_mlp, b_mlp, prelu_a, bn_gamma, bn_beta, W_out, b_out):
    raise NotImplementedError("write your pallas kernel here")


# speedup vs baseline: 184.4940x; 184.4940x over previous
import dataclasses
import functools

import jax
import jax.numpy as jnp
from jax import lax
from jax.experimental import pallas as pl
from jax.experimental.pallas import tpu as pltpu
from jax.experimental.pallas import tpu_sc as plsc

_N = 10000
_NPAD = 10240
_E = 160000
_B = 8
_NSUB = 16
_EPS = _E // _NSUB
_SLICE = _NPAD // _NSUB
_H = 256
_MLPT = 3200
_NT = _MLPT // 8
_GRID = 80000 // _MLPT

_V = 16


def _f(v):
    return jnp.full((_V,), v, jnp.float32)


def _zero_ref(ref):
    zf = _f(0.0)

    @pl.loop(0, ref.shape[0], step=4 * _V)
    def _(i):
        for k in range(4):
            ref[pl.ds(i + k * _V, _V)] = zf


def _vec_add(dst, src, n):
    @pl.loop(0, n, step=_V)
    def _(i):
        dst[pl.ds(i, _V)] += src[pl.ds(i, _V)]


def _sc_gcn_body(src_hbm, dst_hbm, z_hbm, u_hbm, q_hbm,
                 src_c, dst_c, nrm_c, tmp_a, tmp_b, dv_sl,
                 z0, z1, z2, z3, a0, a1, a2, a3,
                 p_sh, a_sh, dv_sh):
    zrefs = (z0, z1, z2, z3)
    arefs = (a0, a1, a2, a3)
    c = lax.axis_index("c")
    s = lax.axis_index("s")
    ebase = s * _EPS
    nbase = s * _SLICE

    pltpu.sync_copy(src_hbm.at[pl.ds(ebase, _EPS)], src_c)
    pltpu.sync_copy(dst_hbm.at[pl.ds(ebase, _EPS)], dst_c)
    for b in range(4):
        pltpu.sync_copy(z_hbm.at[pl.ds((c * 4 + b) * _NPAD, _NPAD)], zrefs[b])

    _zero_ref(a0)
    ones = _f(1.0)

    @pl.loop(0, _EPS, step=_V)
    def _(i):
        d16 = dst_c[pl.ds(i, _V)]
        plsc.addupdate_scatter(a0, [d16], ones)

    pltpu.sync_copy(a0, p_sh.at[pl.ds(s * _NPAD, _NPAD)])
    plsc.subcore_barrier()

    pltpu.sync_copy(p_sh.at[pl.ds(nbase, _SLICE)], tmp_a)
    for j in range(1, _NSUB):
        pltpu.sync_copy(p_sh.at[pl.ds(j * _NPAD + nbase, _SLICE)], tmp_b)
        _vec_add(tmp_a, tmp_b, _SLICE)

    magic = jnp.full((_V,), 0x5F3759DF, jnp.int32)
    c15 = _f(1.5)
    ch = _f(0.5)

    @pl.loop(0, _SLICE, step=_V)
    def _(i):
        x = tmp_a[pl.ds(i, _V)] + ones
        xi = lax.bitcast_convert_type(x, jnp.int32)
        y = lax.bitcast_convert_type(magic - lax.shift_right_logical(xi, 1),
                                     jnp.float32)
        hx = ch * x
        y = y * (c15 - hx * y * y)
        y = y * (c15 - hx * y * y)
        y = y * (c15 - hx * y * y)
        tmp_a[pl.ds(i, _V)] = y

    pltpu.sync_copy(tmp_a, dv_sh.at[pl.ds(nbase, _SLICE)])
    plsc.subcore_barrier()
    pltpu.sync_copy(dv_sh, a0)
    pltpu.sync_copy(dv_sh.at[pl.ds(nbase, _SLICE)], dv_sl)

    @pl.loop(0, _EPS, step=_V)
    def _(i):
        s16 = src_c[pl.ds(i, _V)]
        d16 = dst_c[pl.ds(i, _V)]
        nv = plsc.load_gather(a0, [s16]) * plsc.load_gather(a0, [d16])
        nrm_c[pl.ds(i, _V)] = nv

    _zero_ref(a1)

    @pl.loop(0, _EPS, step=_V)
    def _(i):
        d16 = dst_c[pl.ds(i, _V)]
        plsc.addupdate_scatter(a1, [d16], nrm_c[pl.ds(i, _V)])

    pltpu.sync_copy(a1, p_sh.at[pl.ds(s * _NPAD, _NPAD)])
    plsc.subcore_barrier()

    pltpu.sync_copy(p_sh.at[pl.ds(nbase, _SLICE)], tmp_a)
    for j in range(1, _NSUB):
        pltpu.sync_copy(p_sh.at[pl.ds(j * _NPAD + nbase, _SLICE)], tmp_b)
        _vec_add(tmp_a, tmp_b, _SLICE)

    @pl.loop(0, _SLICE, step=_V)
    def _(i):
        dv = dv_sl[pl.ds(i, _V)]
        tmp_a[pl.ds(i, _V)] += dv * dv

    pltpu.sync_copy(tmp_a, q_hbm.at[pl.ds(c * _NPAD + nbase, _SLICE)])
    plsc.subcore_barrier()

    for layer in range(2):
        for b in range(4):
            _zero_ref(arefs[b])
        for b in range(4):
            zr = zrefs[b]
            ar = arefs[b]

            @pl.loop(0, _EPS, step=_V)
            def _(i, zr=zr, ar=ar):
                s16 = src_c[pl.ds(i, _V)]
                d16 = dst_c[pl.ds(i, _V)]
                zv = plsc.load_gather(zr, [s16])
                plsc.addupdate_scatter(ar, [d16], zv * nrm_c[pl.ds(i, _V)])

        for b in range(4):
            pltpu.sync_copy(arefs[b], p_sh.at[pl.ds(s * _NPAD, _NPAD)])
            plsc.subcore_barrier()

            pltpu.sync_copy(p_sh.at[pl.ds(nbase, _SLICE)], tmp_a)
            for j in range(1, _NSUB):
                pltpu.sync_copy(p_sh.at[pl.ds(j * _NPAD + nbase, _SLICE)],
                                tmp_b)
                _vec_add(tmp_a, tmp_b, _SLICE)

            zr = zrefs[b]

            @pl.loop(0, _SLICE, step=_V)
            def _(i, zr=zr):
                dv = dv_sl[pl.ds(i, _V)]
                tmp_a[pl.ds(i, _V)] += dv * dv * zr[pl.ds(nbase + i, _V)]

            if layer == 0:
                pltpu.sync_copy(tmp_a, a_sh.at[pl.ds(b * _NPAD + nbase, _SLICE)])
            else:
                pltpu.sync_copy(
                    tmp_a,
                    u_hbm.at[pl.ds((c * 4 + b) * _NPAD + nbase, _SLICE)])
            plsc.subcore_barrier()

        if layer == 0:
            for b in range(4):
                pltpu.sync_copy(a_sh.at[pl.ds(b * _NPAD, _NPAD)], zrefs[b])


def _sc_cp():
    cp = pltpu.CompilerParams()
    if "needs_layout_passes" in pltpu.CompilerParams.__dataclass_fields__:
        cp = dataclasses.replace(cp, needs_layout_passes=False)
    return cp


def _sc_gcn(src, dst, z8flat):
    fn = functools.partial(
        pl.kernel,
        compiler_params=_sc_cp(),
        out_type=[jax.ShapeDtypeStruct((_B * _NPAD,), jnp.float32),
                  jax.ShapeDtypeStruct((2 * _NPAD,), jnp.float32)],
        mesh=plsc.VectorSubcoreMesh(core_axis_name="c", subcore_axis_name="s"),
        scratch_types=[
            pltpu.VMEM((_EPS,), jnp.int32),
            pltpu.VMEM((_EPS,), jnp.int32),
            pltpu.VMEM((_EPS,), jnp.float32),
            pltpu.VMEM((_SLICE,), jnp.float32),
            pltpu.VMEM((_SLICE,), jnp.float32),
            pltpu.VMEM((_SLICE,), jnp.float32),
            pltpu.VMEM((_NPAD,), jnp.float32),
            pltpu.VMEM((_NPAD,), jnp.float32),
            pltpu.VMEM((_NPAD,), jnp.float32),
            pltpu.VMEM((_NPAD,), jnp.float32),
            pltpu.VMEM((_NPAD,), jnp.float32),
            pltpu.VMEM((_NPAD,), jnp.float32),
            pltpu.VMEM((_NPAD,), jnp.float32),
            pltpu.VMEM((_NPAD,), jnp.float32),
            pltpu.VMEM_SHARED((_NSUB * _NPAD,), jnp.float32),
            pltpu.VMEM_SHARED((4 * _NPAD,), jnp.float32),
            pltpu.VMEM_SHARED((_NPAD,), jnp.float32),
        ],
    )(_sc_gcn_body)
    return fn(src, dst, z8flat)


def _gru_body(x_ref, wih_ref, bih_ref, bhh_ref, wg1_ref, bg1_ref,
              wg2_ref, bg2_ref, woutt_ref, bout_ref, h_ref, c_ref):
    gi = lax.dot_general(x_ref[...], wih_ref[...], (((1,), (1,)), ((), ())),
                         preferred_element_type=jnp.float32,
                         precision=lax.Precision.HIGHEST)
    gi = gi + bih_ref[...]
    bhh = bhh_ref[...]
    r = jax.nn.sigmoid(gi[:, :_H] + bhh[:, :_H])
    zg = jax.nn.sigmoid(gi[:, _H:2 * _H] + bhh[:, _H:2 * _H])
    n = jnp.tanh(gi[:, 2 * _H:] + r * bhh[:, 2 * _H:])
    h_ref[...] = (1.0 - zg) * n

    w_r = woutt_ref[:, 8:]
    v = lax.dot_general(w_r, wg2_ref[...], (((1,), (1,)), ((), ())),
                        preferred_element_type=jnp.float32,
                        precision=lax.Precision.HIGHEST)
    c1 = jnp.sum(wg1_ref[...] * v)
    c2 = jnp.sum(bg1_ref[...] * v)
    c3 = jnp.sum(bg2_ref[...] * w_r) + bout_ref[0, 0]
    idx = lax.broadcasted_iota(jnp.int32, (1, 128), 1)
    c_ref[...] = (jnp.where(idx == 0, c1, 0.0) +
                  jnp.where(idx == 1, c2, 0.0) +
                  jnp.where(idx == 2, c3, 0.0))


def _gru(x, W_ih, b_ih, b_hh, W_gcn1, b_gcn1, W_gcn2, b_gcn2, W_out, b_out):
    return pl.pallas_call(
        _gru_body,
        out_shape=[jax.ShapeDtypeStruct((_B, _H), jnp.float32),
                   jax.ShapeDtypeStruct((1, 128), jnp.float32)],
    )(x, W_ih, b_ih.reshape(1, -1), b_hh.reshape(1, -1), W_gcn1,
      b_gcn1.reshape(1, -1), W_gcn2, b_gcn2.reshape(1, -1),
      W_out.T, b_out.reshape(1, 1))


_BN_SCALE = (1.0 + 1e-5) ** -0.5


def _mlp_body(h_ref, wm_ref, bm_ref, pa_ref, g_ref, be_ref, s2_ref, o_ref):
    m = lax.dot_general(h_ref[...], wm_ref[...], (((1,), (0,)), ((), ())),
                        preferred_element_type=jnp.float32,
                        precision=lax.Precision.HIGHEST)
    m = m + bm_ref[...]
    m = jnp.where(m >= 0, m, pa_ref[...] * m)
    m = g_ref[...] * (m * _BN_SCALE) + be_ref[...]
    y = lax.dot_general(m, s2_ref[...], (((1,), (0,)), ((), ())),
                        preferred_element_type=jnp.float32,
                        precision=lax.Precision.HIGHEST)
    o_ref[...] = y.reshape(1, _B, _NT)


def _mlp(h, W_mlp, b_mlp, prelu_a, bn_gamma, bn_beta, s2):
    row = lambda a: a.reshape(1, -1)
    return pl.pallas_call(
        _mlp_body,
        grid=(_GRID,),
        in_specs=[
            pl.BlockSpec((_B, _H), lambda i: (0, 0)),
            pl.BlockSpec((_H, _MLPT), lambda i: (0, i)),
            pl.BlockSpec((1, _MLPT), lambda i: (0, i)),
            pl.BlockSpec((1, _MLPT), lambda i: (0, i)),
            pl.BlockSpec((1, _MLPT), lambda i: (0, i)),
            pl.BlockSpec((1, _MLPT), lambda i: (0, i)),
            pl.BlockSpec((_MLPT, _NT), lambda i: (0, 0)),
        ],
        out_specs=pl.BlockSpec((1, _B, _NT), lambda i: (i, 0, 0)),
        out_shape=jax.ShapeDtypeStruct((_GRID, _B, _NT), jnp.float32),
    )(h, W_mlp, row(b_mlp), row(prelu_a), row(bn_gamma), row(bn_beta), s2)


def _join_body(yg_ref, u_ref, q_ref, c_ref, o_ref):
    c1 = c_ref[0, 0]
    c2 = c_ref[0, 1]
    c3 = c_ref[0, 2]
    o_ref[...] = yg_ref[...] + c1 * u_ref[...] + c2 * q_ref[...] + c3


def _join(ygru, u, q, cvec):
    return pl.pallas_call(
        _join_body,
        out_shape=jax.ShapeDtypeStruct((_B, _N), jnp.float32),
    )(ygru, u, q, cvec)


def kernel(x, smoothed_vert_pos, edge_index, W_gcn1, b_gcn1, W_gcn2, b_gcn2,
           W_ih, W_hh, b_ih, b_hh, W_mlp, b_mlp, prelu_a, bn_gamma, bn_beta,
           W_out, b_out):
    z8 = smoothed_vert_pos.reshape(_B, _N, 3)[:, :, 2]
    z8p = jnp.pad(z8, ((0, 0), (0, _NPAD - _N)))
    u8f, q2f = _sc_gcn(edge_index[0], edge_index[1], z8p.reshape(-1))
    u8 = u8f.reshape(_B, _NPAD)
    q2 = q2f.reshape(2, _NPAD)
    h, cvec = _gru(x, W_ih, b_ih, b_hh, W_gcn1, b_gcn1, W_gcn2, b_gcn2,
                   W_out, b_out)
    s2 = jnp.kron(jnp.eye(_NT, dtype=jnp.float32), W_out[:8, 0:1])
    ygru = _mlp(h, W_mlp, b_mlp, prelu_a, bn_gamma, bn_beta, s2)
    ygru = ygru.transpose(1, 0, 2).reshape(_B, _N)
    y = _join(ygru, u8[:, :_N], q2[0:1, :_N], cvec)
    return y, h


# speedup vs baseline: 200.7503x; 1.0881x over previous
import dataclasses
import functools

import jax
import jax.numpy as jnp
from jax import lax
from jax.experimental import pallas as pl
from jax.experimental.pallas import tpu as pltpu
from jax.experimental.pallas import tpu_sc as plsc

_N = 10000
_NPAD = 10240
_E = 160000
_B = 8
_NSUB = 16
_EPS = _E // _NSUB
_SLICE = _NPAD // _NSUB
_H = 256
_MLPT = 3200
_NT = _MLPT // 8
_GRID = 80000 // _MLPT

_V = 16


def _f(v):
    return jnp.full((_V,), v, jnp.float32)


def _zero_ref(ref):
    zf = _f(0.0)

    @pl.loop(0, ref.shape[0], step=4 * _V)
    def _(i):
        for k in range(4):
            ref[pl.ds(i + k * _V, _V)] = zf


def _vec_add(dst, src, n):
    @pl.loop(0, n, step=_V)
    def _(i):
        dst[pl.ds(i, _V)] += src[pl.ds(i, _V)]


_EMAIN = (_EPS // (2 * _V)) * (2 * _V)


def _edge_loop(body):
    @pl.loop(0, _EMAIN, step=2 * _V)
    def _(i):
        body(i)
        body(i + _V)

    for o in range(_EMAIN, _EPS, _V):
        body(o)


def _sc_gcn_body(src_hbm, dst_hbm, z_hbm, u_hbm, q_hbm,
                 src_c, dst_c, nrm_c, tmp_a, tmp_b, dv_sl,
                 z0, z1, z2, z3, a0, a1, a2, a3,
                 p_sh, a_sh, dv_sh):
    zrefs = (z0, z1, z2, z3)
    arefs = (a0, a1, a2, a3)
    c = lax.axis_index("c")
    s = lax.axis_index("s")
    ebase = s * _EPS
    nbase = s * _SLICE

    pltpu.sync_copy(src_hbm.at[pl.ds(ebase, _EPS)], src_c)
    pltpu.sync_copy(dst_hbm.at[pl.ds(ebase, _EPS)], dst_c)
    for b in range(4):
        pltpu.sync_copy(z_hbm.at[pl.ds((c * 4 + b) * _NPAD, _NPAD)], zrefs[b])

    _zero_ref(a0)
    ones = _f(1.0)

    def _deg_body(o):
        d16 = dst_c[pl.ds(o, _V)]
        plsc.addupdate_scatter(a0, [d16], ones)

    _edge_loop(_deg_body)

    pltpu.sync_copy(a0, p_sh.at[pl.ds(s * _NPAD, _NPAD)])
    plsc.subcore_barrier()

    pltpu.sync_copy(p_sh.at[pl.ds(nbase, _SLICE)], tmp_a)
    for j in range(1, _NSUB):
        pltpu.sync_copy(p_sh.at[pl.ds(j * _NPAD + nbase, _SLICE)], tmp_b)
        _vec_add(tmp_a, tmp_b, _SLICE)

    magic = jnp.full((_V,), 0x5F3759DF, jnp.int32)
    c15 = _f(1.5)
    ch = _f(0.5)

    @pl.loop(0, _SLICE, step=_V)
    def _(i):
        x = tmp_a[pl.ds(i, _V)] + ones
        xi = lax.bitcast_convert_type(x, jnp.int32)
        y = lax.bitcast_convert_type(magic - lax.shift_right_logical(xi, 1),
                                     jnp.float32)
        hx = ch * x
        y = y * (c15 - hx * y * y)
        y = y * (c15 - hx * y * y)
        y = y * (c15 - hx * y * y)
        tmp_a[pl.ds(i, _V)] = y

    pltpu.sync_copy(tmp_a, dv_sh.at[pl.ds(nbase, _SLICE)])
    plsc.subcore_barrier()
    pltpu.sync_copy(dv_sh, a0)
    pltpu.sync_copy(dv_sh.at[pl.ds(nbase, _SLICE)], dv_sl)

    _zero_ref(a1)

    def _norm_body(o):
        s16 = src_c[pl.ds(o, _V)]
        d16 = dst_c[pl.ds(o, _V)]
        nv = plsc.load_gather(a0, [s16]) * plsc.load_gather(a0, [d16])
        nrm_c[pl.ds(o, _V)] = nv
        plsc.addupdate_scatter(a1, [d16], nv)

    _edge_loop(_norm_body)

    pltpu.sync_copy(a1, p_sh.at[pl.ds(s * _NPAD, _NPAD)])
    plsc.subcore_barrier()

    pltpu.sync_copy(p_sh.at[pl.ds(nbase, _SLICE)], tmp_a)
    for j in range(1, _NSUB):
        pltpu.sync_copy(p_sh.at[pl.ds(j * _NPAD + nbase, _SLICE)], tmp_b)
        _vec_add(tmp_a, tmp_b, _SLICE)

    @pl.loop(0, _SLICE, step=_V)
    def _(i):
        dv = dv_sl[pl.ds(i, _V)]
        tmp_a[pl.ds(i, _V)] += dv * dv

    pltpu.sync_copy(tmp_a, q_hbm.at[pl.ds(c * _NPAD + nbase, _SLICE)])
    plsc.subcore_barrier()

    for layer in range(2):
        for b in range(4):
            _zero_ref(arefs[b])

        def _msg_body(o):
            s16 = src_c[pl.ds(o, _V)]
            d16 = dst_c[pl.ds(o, _V)]
            n16 = nrm_c[pl.ds(o, _V)]
            for b in range(4):
                zv = plsc.load_gather(zrefs[b], [s16])
                plsc.addupdate_scatter(arefs[b], [d16], zv * n16)

        _edge_loop(_msg_body)

        for b in range(4):
            pltpu.sync_copy(arefs[b], p_sh.at[pl.ds(s * _NPAD, _NPAD)])
            plsc.subcore_barrier()

            pltpu.sync_copy(p_sh.at[pl.ds(nbase, _SLICE)], tmp_a)
            for j in range(1, _NSUB):
                pltpu.sync_copy(p_sh.at[pl.ds(j * _NPAD + nbase, _SLICE)],
                                tmp_b)
                _vec_add(tmp_a, tmp_b, _SLICE)

            zr = zrefs[b]

            @pl.loop(0, _SLICE, step=_V)
            def _(i, zr=zr):
                dv = dv_sl[pl.ds(i, _V)]
                tmp_a[pl.ds(i, _V)] += dv * dv * zr[pl.ds(nbase + i, _V)]

            if layer == 0:
                pltpu.sync_copy(tmp_a, a_sh.at[pl.ds(b * _NPAD + nbase, _SLICE)])
            else:
                pltpu.sync_copy(
                    tmp_a,
                    u_hbm.at[pl.ds((c * 4 + b) * _NPAD + nbase, _SLICE)])
            plsc.subcore_barrier()

        if layer == 0:
            for b in range(4):
                pltpu.sync_copy(a_sh.at[pl.ds(b * _NPAD, _NPAD)], zrefs[b])


def _sc_cp():
    cp = pltpu.CompilerParams()
    if "needs_layout_passes" in pltpu.CompilerParams.__dataclass_fields__:
        cp = dataclasses.replace(cp, needs_layout_passes=False)
    return cp


def _sc_gcn(src, dst, z8flat):
    fn = functools.partial(
        pl.kernel,
        compiler_params=_sc_cp(),
        out_type=[jax.ShapeDtypeStruct((_B * _NPAD,), jnp.float32),
                  jax.ShapeDtypeStruct((2 * _NPAD,), jnp.float32)],
        mesh=plsc.VectorSubcoreMesh(core_axis_name="c", subcore_axis_name="s"),
        scratch_types=[
            pltpu.VMEM((_EPS,), jnp.int32),
            pltpu.VMEM((_EPS,), jnp.int32),
            pltpu.VMEM((_EPS,), jnp.float32),
            pltpu.VMEM((_SLICE,), jnp.float32),
            pltpu.VMEM((_SLICE,), jnp.float32),
            pltpu.VMEM((_SLICE,), jnp.float32),
            pltpu.VMEM((_NPAD,), jnp.float32),
            pltpu.VMEM((_NPAD,), jnp.float32),
            pltpu.VMEM((_NPAD,), jnp.float32),
            pltpu.VMEM((_NPAD,), jnp.float32),
            pltpu.VMEM((_NPAD,), jnp.float32),
            pltpu.VMEM((_NPAD,), jnp.float32),
            pltpu.VMEM((_NPAD,), jnp.float32),
            pltpu.VMEM((_NPAD,), jnp.float32),
            pltpu.VMEM_SHARED((_NSUB * _NPAD,), jnp.float32),
            pltpu.VMEM_SHARED((4 * _NPAD,), jnp.float32),
            pltpu.VMEM_SHARED((_NPAD,), jnp.float32),
        ],
    )(_sc_gcn_body)
    return fn(src, dst, z8flat)


def _gru_body(x_ref, wih_ref, bih_ref, bhh_ref, wg1_ref, bg1_ref,
              wg2_ref, bg2_ref, woutt_ref, bout_ref, h_ref, c_ref):
    gi = lax.dot_general(x_ref[...], wih_ref[...], (((1,), (1,)), ((), ())),
                         preferred_element_type=jnp.float32,
                         precision=lax.Precision.HIGHEST)
    gi = gi + bih_ref[...]
    bhh = bhh_ref[...]
    r = jax.nn.sigmoid(gi[:, :_H] + bhh[:, :_H])
    zg = jax.nn.sigmoid(gi[:, _H:2 * _H] + bhh[:, _H:2 * _H])
    n = jnp.tanh(gi[:, 2 * _H:] + r * bhh[:, 2 * _H:])
    h_ref[...] = (1.0 - zg) * n

    w_r = woutt_ref[:, 8:]
    v = lax.dot_general(w_r, wg2_ref[...], (((1,), (1,)), ((), ())),
                        preferred_element_type=jnp.float32,
                        precision=lax.Precision.HIGHEST)
    c1 = jnp.sum(wg1_ref[...] * v)
    c2 = jnp.sum(bg1_ref[...] * v)
    c3 = jnp.sum(bg2_ref[...] * w_r) + bout_ref[0, 0]
    idx = lax.broadcasted_iota(jnp.int32, (1, 128), 1)
    c_ref[...] = (jnp.where(idx == 0, c1, 0.0) +
                  jnp.where(idx == 1, c2, 0.0) +
                  jnp.where(idx == 2, c3, 0.0))


def _gru(x, W_ih, b_ih, b_hh, W_gcn1, b_gcn1, W_gcn2, b_gcn2, W_out, b_out):
    return pl.pallas_call(
        _gru_body,
        out_shape=[jax.ShapeDtypeStruct((_B, _H), jnp.float32),
                   jax.ShapeDtypeStruct((1, 128), jnp.float32)],
    )(x, W_ih, b_ih.reshape(1, -1), b_hh.reshape(1, -1), W_gcn1,
      b_gcn1.reshape(1, -1), W_gcn2, b_gcn2.reshape(1, -1),
      W_out.T, b_out.reshape(1, 1))


_BN_SCALE = (1.0 + 1e-5) ** -0.5


def _mlp_body(h_ref, wm_ref, bm_ref, pa_ref, g_ref, be_ref, s2_ref, o_ref):
    m = lax.dot_general(h_ref[...], wm_ref[...], (((1,), (0,)), ((), ())),
                        preferred_element_type=jnp.float32,
                        precision=lax.Precision.HIGHEST)
    m = m + bm_ref[...]
    m = jnp.where(m >= 0, m, pa_ref[...] * m)
    m = g_ref[...] * (m * _BN_SCALE) + be_ref[...]
    y = lax.dot_general(m, s2_ref[...], (((1,), (0,)), ((), ())),
                        preferred_element_type=jnp.float32,
                        precision=lax.Precision.HIGHEST)
    o_ref[...] = y.reshape(1, _B, _NT)


def _mlp(h, W_mlp, b_mlp, prelu_a, bn_gamma, bn_beta, s2):
    row = lambda a: a.reshape(1, -1)
    return pl.pallas_call(
        _mlp_body,
        grid=(_GRID,),
        in_specs=[
            pl.BlockSpec((_B, _H), lambda i: (0, 0)),
            pl.BlockSpec((_H, _MLPT), lambda i: (0, i)),
            pl.BlockSpec((1, _MLPT), lambda i: (0, i)),
            pl.BlockSpec((1, _MLPT), lambda i: (0, i)),
            pl.BlockSpec((1, _MLPT), lambda i: (0, i)),
            pl.BlockSpec((1, _MLPT), lambda i: (0, i)),
            pl.BlockSpec((_MLPT, _NT), lambda i: (0, 0)),
        ],
        out_specs=pl.BlockSpec((1, _B, _NT), lambda i: (i, 0, 0)),
        out_shape=jax.ShapeDtypeStruct((_GRID, _B, _NT), jnp.float32),
    )(h, W_mlp, row(b_mlp), row(prelu_a), row(bn_gamma), row(bn_beta), s2)


def _join_body(yg_ref, u_ref, q_ref, c_ref, o_ref):
    c1 = c_ref[0, 0]
    c2 = c_ref[0, 1]
    c3 = c_ref[0, 2]
    o_ref[...] = yg_ref[...] + c1 * u_ref[...] + c2 * q_ref[...] + c3


def _join(ygru, u, q, cvec):
    return pl.pallas_call(
        _join_body,
        out_shape=jax.ShapeDtypeStruct((_B, _N), jnp.float32),
    )(ygru, u, q, cvec)


def kernel(x, smoothed_vert_pos, edge_index, W_gcn1, b_gcn1, W_gcn2, b_gcn2,
           W_ih, W_hh, b_ih, b_hh, W_mlp, b_mlp, prelu_a, bn_gamma, bn_beta,
           W_out, b_out):
    z8 = smoothed_vert_pos.reshape(_B, _N, 3)[:, :, 2]
    z8p = jnp.pad(z8, ((0, 0), (0, _NPAD - _N)))
    u8f, q2f = _sc_gcn(edge_index[0], edge_index[1], z8p.reshape(-1))
    u8 = u8f.reshape(_B, _NPAD)
    q2 = q2f.reshape(2, _NPAD)
    h, cvec = _gru(x, W_ih, b_ih, b_hh, W_gcn1, b_gcn1, W_gcn2, b_gcn2,
                   W_out, b_out)
    s2 = jnp.kron(jnp.eye(_NT, dtype=jnp.float32), W_out[:8, 0:1])
    ygru = _mlp(h, W_mlp, b_mlp, prelu_a, bn_gamma, bn_beta, s2)
    ygru = ygru.transpose(1, 0, 2).reshape(_B, _N)
    y = _join(ygru, u8[:, :_N], q2[0:1, :_N], cvec)
    return y, h


# speedup vs baseline: 200.7950x; 1.0002x over previous
import dataclasses
import functools

import jax
import jax.numpy as jnp
from jax import lax
from jax.experimental import pallas as pl
from jax.experimental.pallas import tpu as pltpu
from jax.experimental.pallas import tpu_sc as plsc

_N = 10000
_NPAD = 10240
_E = 160000
_B = 8
_NSUB = 16
_EPS = _E // _NSUB
_SLICE = _NPAD // _NSUB
_H = 256
_MLPT = 3200
_NT = _MLPT // 8
_GRID = 80000 // _MLPT

_V = 16


def _f(v):
    return jnp.full((_V,), v, jnp.float32)


def _zero_ref(ref):
    zf = _f(0.0)

    @pl.loop(0, ref.shape[0], step=4 * _V)
    def _(i):
        for k in range(4):
            ref[pl.ds(i + k * _V, _V)] = zf


def _vec_add(dst, src, n):
    @pl.loop(0, n, step=_V)
    def _(i):
        dst[pl.ds(i, _V)] += src[pl.ds(i, _V)]


_EMAIN = (_EPS // (2 * _V)) * (2 * _V)


def _edge_loop(body):
    @pl.loop(0, _EMAIN, step=2 * _V)
    def _(i):
        body(i)
        body(i + _V)

    for o in range(_EMAIN, _EPS, _V):
        body(o)


def _sc_gcn_body(src_hbm, dst_hbm, z_hbm, u_hbm, q_hbm,
                 src_c, dst_c, nrm_c, tmp_a, tmp_b, dv_sl,
                 z0, z1, z2, z3, a0, a1, a2, a3,
                 p_sh, a_sh, dv_sh):
    zrefs = (z0, z1, z2, z3)
    arefs = (a0, a1, a2, a3)
    c = lax.axis_index("c")
    s = lax.axis_index("s")
    ebase = s * _EPS
    nbase = s * _SLICE

    pltpu.sync_copy(src_hbm.at[pl.ds(ebase, _EPS)], src_c)
    pltpu.sync_copy(dst_hbm.at[pl.ds(ebase, _EPS)], dst_c)
    for b in range(4):
        pltpu.sync_copy(z_hbm.at[pl.ds((c * 4 + b) * _NPAD, _NPAD)], zrefs[b])

    _zero_ref(a0)
    ones = _f(1.0)

    def _deg_body(o):
        d16 = dst_c[pl.ds(o, _V)]
        plsc.addupdate_scatter(a0, [d16], ones)

    _edge_loop(_deg_body)

    pltpu.sync_copy(a0, p_sh.at[pl.ds(s * _NPAD, _NPAD)])
    plsc.subcore_barrier()

    pltpu.sync_copy(p_sh.at[pl.ds(nbase, _SLICE)], tmp_a)
    for j in range(1, _NSUB):
        pltpu.sync_copy(p_sh.at[pl.ds(j * _NPAD + nbase, _SLICE)], tmp_b)
        _vec_add(tmp_a, tmp_b, _SLICE)

    magic = jnp.full((_V,), 0x5F3759DF, jnp.int32)
    c15 = _f(1.5)
    ch = _f(0.5)

    @pl.loop(0, _SLICE, step=_V)
    def _(i):
        x = tmp_a[pl.ds(i, _V)] + ones
        xi = lax.bitcast_convert_type(x, jnp.int32)
        y = lax.bitcast_convert_type(magic - lax.shift_right_logical(xi, 1),
                                     jnp.float32)
        hx = ch * x
        y = y * (c15 - hx * y * y)
        y = y * (c15 - hx * y * y)
        y = y * (c15 - hx * y * y)
        tmp_a[pl.ds(i, _V)] = y

    pltpu.sync_copy(tmp_a, dv_sh.at[pl.ds(nbase, _SLICE)])
    plsc.subcore_barrier()
    pltpu.sync_copy(dv_sh, a0)
    pltpu.sync_copy(dv_sh.at[pl.ds(nbase, _SLICE)], dv_sl)

    _zero_ref(a1)

    def _norm_body(o):
        s16 = src_c[pl.ds(o, _V)]
        d16 = dst_c[pl.ds(o, _V)]
        nv = plsc.load_gather(a0, [s16]) * plsc.load_gather(a0, [d16])
        nrm_c[pl.ds(o, _V)] = nv
        plsc.addupdate_scatter(a1, [d16], nv)

    _edge_loop(_norm_body)

    pltpu.sync_copy(a1, p_sh.at[pl.ds(s * _NPAD, _NPAD)])
    plsc.subcore_barrier()

    pltpu.sync_copy(p_sh.at[pl.ds(nbase, _SLICE)], tmp_a)
    for j in range(1, _NSUB):
        pltpu.sync_copy(p_sh.at[pl.ds(j * _NPAD + nbase, _SLICE)], tmp_b)
        _vec_add(tmp_a, tmp_b, _SLICE)

    @pl.loop(0, _SLICE, step=_V)
    def _(i):
        dv = dv_sl[pl.ds(i, _V)]
        tmp_a[pl.ds(i, _V)] += dv * dv

    pltpu.sync_copy(tmp_a, q_hbm.at[pl.ds(c * _NPAD + nbase, _SLICE)])
    plsc.subcore_barrier()

    for layer in range(2):
        for b in range(4):
            _zero_ref(arefs[b])

        def _msg_body(o):
            s16 = src_c[pl.ds(o, _V)]
            d16 = dst_c[pl.ds(o, _V)]
            n16 = nrm_c[pl.ds(o, _V)]
            for b in range(4):
                zv = plsc.load_gather(zrefs[b], [s16])
                plsc.addupdate_scatter(arefs[b], [d16], zv * n16)

        _edge_loop(_msg_body)

        for b in range(4):
            pltpu.sync_copy(arefs[b], p_sh.at[pl.ds(s * _NPAD, _NPAD)])
            plsc.subcore_barrier()

            pltpu.sync_copy(p_sh.at[pl.ds(nbase, _SLICE)], tmp_a)
            for j in range(1, _NSUB):
                pltpu.sync_copy(p_sh.at[pl.ds(j * _NPAD + nbase, _SLICE)],
                                tmp_b)
                _vec_add(tmp_a, tmp_b, _SLICE)

            zr = zrefs[b]

            @pl.loop(0, _SLICE, step=_V)
            def _(i, zr=zr):
                dv = dv_sl[pl.ds(i, _V)]
                tmp_a[pl.ds(i, _V)] += dv * dv * zr[pl.ds(nbase + i, _V)]

            if layer == 0:
                pltpu.sync_copy(tmp_a, a_sh.at[pl.ds(b * _NPAD + nbase, _SLICE)])
            else:
                pltpu.sync_copy(
                    tmp_a,
                    u_hbm.at[pl.ds((c * 4 + b) * _NPAD + nbase, _SLICE)])
            plsc.subcore_barrier()

        if layer == 0:
            for b in range(4):
                pltpu.sync_copy(a_sh.at[pl.ds(b * _NPAD, _NPAD)], zrefs[b])


def _sc_cp():
    cp = pltpu.CompilerParams()
    if "needs_layout_passes" in pltpu.CompilerParams.__dataclass_fields__:
        cp = dataclasses.replace(cp, needs_layout_passes=False)
    return cp


def _sc_gcn(src, dst, z8flat):
    fn = functools.partial(
        pl.kernel,
        compiler_params=_sc_cp(),
        out_type=[jax.ShapeDtypeStruct((_B * _NPAD,), jnp.float32),
                  jax.ShapeDtypeStruct((2 * _NPAD,), jnp.float32)],
        mesh=plsc.VectorSubcoreMesh(core_axis_name="c", subcore_axis_name="s"),
        scratch_types=[
            pltpu.VMEM((_EPS,), jnp.int32),
            pltpu.VMEM((_EPS,), jnp.int32),
            pltpu.VMEM((_EPS,), jnp.float32),
            pltpu.VMEM((_SLICE,), jnp.float32),
            pltpu.VMEM((_SLICE,), jnp.float32),
            pltpu.VMEM((_SLICE,), jnp.float32),
            pltpu.VMEM((_NPAD,), jnp.float32),
            pltpu.VMEM((_NPAD,), jnp.float32),
            pltpu.VMEM((_NPAD,), jnp.float32),
            pltpu.VMEM((_NPAD,), jnp.float32),
            pltpu.VMEM((_NPAD,), jnp.float32),
            pltpu.VMEM((_NPAD,), jnp.float32),
            pltpu.VMEM((_NPAD,), jnp.float32),
            pltpu.VMEM((_NPAD,), jnp.float32),
            pltpu.VMEM_SHARED((_NSUB * _NPAD,), jnp.float32),
            pltpu.VMEM_SHARED((4 * _NPAD,), jnp.float32),
            pltpu.VMEM_SHARED((_NPAD,), jnp.float32),
        ],
    )(_sc_gcn_body)
    return fn(src, dst, z8flat)


def _gru_body(x_ref, wih_ref, bih_ref, bhh_ref, wg1_ref, bg1_ref,
              wg2_ref, bg2_ref, woutt_ref, bout_ref, h_ref, c_ref):
    gi = lax.dot_general(x_ref[...], wih_ref[...], (((1,), (1,)), ((), ())),
                         preferred_element_type=jnp.float32,
                         precision=lax.Precision.HIGHEST)
    gi = gi + bih_ref[...]
    bhh = bhh_ref[...]
    r = jax.nn.sigmoid(gi[:, :_H] + bhh[:, :_H])
    zg = jax.nn.sigmoid(gi[:, _H:2 * _H] + bhh[:, _H:2 * _H])
    n = jnp.tanh(gi[:, 2 * _H:] + r * bhh[:, 2 * _H:])
    h_ref[...] = (1.0 - zg) * n

    w_r = woutt_ref[:, 8:]
    v = lax.dot_general(w_r, wg2_ref[...], (((1,), (1,)), ((), ())),
                        preferred_element_type=jnp.float32,
                        precision=lax.Precision.HIGHEST)
    c1 = jnp.sum(wg1_ref[...] * v)
    c2 = jnp.sum(bg1_ref[...] * v)
    c3 = jnp.sum(bg2_ref[...] * w_r) + bout_ref[0, 0]
    idx = lax.broadcasted_iota(jnp.int32, (1, 128), 1)
    c_ref[...] = (jnp.where(idx == 0, c1, 0.0) +
                  jnp.where(idx == 1, c2, 0.0) +
                  jnp.where(idx == 2, c3, 0.0))


def _gru(x, W_ih, b_ih, b_hh, W_gcn1, b_gcn1, W_gcn2, b_gcn2, W_out, b_out):
    return pl.pallas_call(
        _gru_body,
        out_shape=[jax.ShapeDtypeStruct((_B, _H), jnp.float32),
                   jax.ShapeDtypeStruct((1, 128), jnp.float32)],
    )(x, W_ih, b_ih.reshape(1, -1), b_hh.reshape(1, -1), W_gcn1,
      b_gcn1.reshape(1, -1), W_gcn2, b_gcn2.reshape(1, -1),
      W_out.T, b_out.reshape(1, 1))


_BN_SCALE = (1.0 + 1e-5) ** -0.5


def _mlp_body(h_ref, wm_ref, bm_ref, pa_ref, g_ref, be_ref, s2_ref, o_ref):
    m = lax.dot_general(h_ref[...], wm_ref[...], (((1,), (0,)), ((), ())),
                        preferred_element_type=jnp.float32)
    m = m + bm_ref[...]
    m = jnp.where(m >= 0, m, pa_ref[...] * m)
    m = g_ref[...] * (m * _BN_SCALE) + be_ref[...]
    y = lax.dot_general(m, s2_ref[...], (((1,), (0,)), ((), ())),
                        preferred_element_type=jnp.float32)
    o_ref[...] = y.reshape(1, _B, _NT)


def _mlp(h, W_mlp, b_mlp, prelu_a, bn_gamma, bn_beta, s2):
    row = lambda a: a.reshape(1, -1)
    return pl.pallas_call(
        _mlp_body,
        grid=(_GRID,),
        in_specs=[
            pl.BlockSpec((_B, _H), lambda i: (0, 0)),
            pl.BlockSpec((_H, _MLPT), lambda i: (0, i)),
            pl.BlockSpec((1, _MLPT), lambda i: (0, i)),
            pl.BlockSpec((1, _MLPT), lambda i: (0, i)),
            pl.BlockSpec((1, _MLPT), lambda i: (0, i)),
            pl.BlockSpec((1, _MLPT), lambda i: (0, i)),
            pl.BlockSpec((_MLPT, _NT), lambda i: (0, 0)),
        ],
        out_specs=pl.BlockSpec((1, _B, _NT), lambda i: (i, 0, 0)),
        out_shape=jax.ShapeDtypeStruct((_GRID, _B, _NT), jnp.float32),
    )(h, W_mlp, row(b_mlp), row(prelu_a), row(bn_gamma), row(bn_beta), s2)


def _join_body(yg_ref, u_ref, q_ref, c_ref, o_ref):
    c1 = c_ref[0, 0]
    c2 = c_ref[0, 1]
    c3 = c_ref[0, 2]
    o_ref[...] = yg_ref[...] + c1 * u_ref[...] + c2 * q_ref[...] + c3


def _join(ygru, u, q, cvec):
    return pl.pallas_call(
        _join_body,
        out_shape=jax.ShapeDtypeStruct((_B, _N), jnp.float32),
    )(ygru, u, q, cvec)


def kernel(x, smoothed_vert_pos, edge_index, W_gcn1, b_gcn1, W_gcn2, b_gcn2,
           W_ih, W_hh, b_ih, b_hh, W_mlp, b_mlp, prelu_a, bn_gamma, bn_beta,
           W_out, b_out):
    z8 = smoothed_vert_pos.reshape(_B, _N, 3)[:, :, 2]
    z8p = jnp.pad(z8, ((0, 0), (0, _NPAD - _N)))
    u8f, q2f = _sc_gcn(edge_index[0], edge_index[1], z8p.reshape(-1))
    u8 = u8f.reshape(_B, _NPAD)
    q2 = q2f.reshape(2, _NPAD)
    h, cvec = _gru(x, W_ih, b_ih, b_hh, W_gcn1, b_gcn1, W_gcn2, b_gcn2,
                   W_out, b_out)
    s2 = jnp.kron(jnp.eye(_NT, dtype=jnp.float32), W_out[:8, 0:1])
    ygru = _mlp(h, W_mlp, b_mlp, prelu_a, bn_gamma, bn_beta, s2)
    ygru = ygru.transpose(1, 0, 2).reshape(_B, _N)
    y = _join(ygru, u8[:, :_N], q2[0:1, :_N], cvec)
    return y, h
